# baseline (speedup 1.0000x reference)
import jax
import jax.numpy as jnp
from jax.experimental import pallas as pl
from jax.experimental.pallas import tpu as pltpu

_B = 8192
_D = 64
_K = 8192
_Q = 3
_BETA = 0.25
_BLK = 256
_NBLK = _B // _BLK
_EPS = 1e-12


def _vq_body(x_ref, cb_ref, cbt_ref, t0_ref, t1_ref, t2_ref, idx_ref,
             loss_ref, et_ref, en_ref):
    i = pl.program_id(0)

    @pl.when(i == 0)
    def _init():
        cb = cb_ref[...]
        nrm = jnp.sqrt(jnp.sum(cb * cb, axis=1, keepdims=True))
        en_ref[...] = cb / jnp.maximum(nrm, _EPS)
        cbt = cbt_ref[...]
        nrm_t = jnp.sqrt(jnp.sum(cbt * cbt, axis=0, keepdims=True))
        et_ref[...] = (cbt / jnp.maximum(nrm_t, _EPS)).astype(jnp.bfloat16)
        loss_ref[...] = jnp.zeros((1, 1), jnp.float32)

    et = et_ref[...]
    r = x_ref[...]
    tok_refs = (t0_ref, t1_ref, t2_ref)
    idx_cols = []
    loss_acc = jnp.zeros((1, 1), jnp.float32)
    for q in range(_Q):
        rn = jnp.sqrt(jnp.sum(r * r, axis=1, keepdims=True))
        n = r / jnp.maximum(rn, _EPS)
        dist = jax.lax.dot_general(
            n.astype(jnp.bfloat16), et, (((1,), (0,)), ((), ())),
            preferred_element_type=jnp.float32)
        idx = jnp.argmax(dist, axis=1, keepdims=True).astype(jnp.int32)
        iota = jax.lax.broadcasted_iota(jnp.int32, (_BLK, _K), 1)
        onehot = (iota == idx).astype(jnp.float32)
        quant = jax.lax.dot_general(
            onehot, en_ref[...], (((1,), (0,)), ((), ())),
            precision=jax.lax.Precision.HIGHEST,
            preferred_element_type=jnp.float32)
        quantized = r + (quant - r)
        qn = jnp.sqrt(jnp.sum(quantized * quantized, axis=1, keepdims=True))
        c = quantized / jnp.maximum(qn, _EPS)
        cos = jnp.clip(jnp.sum(n * c, axis=1, keepdims=True), -1.0, 1.0)
        loss_acc += (_BETA / _B) * jnp.sum(1.0 - cos, keepdims=True)
        tok_refs[q][...] = r + (quantized - r)
        idx_cols.append(idx)
        alpha = jnp.sum(r * c, axis=1, keepdims=True)
        r = r - alpha * c
    loss_acc += jnp.sum(r * r, keepdims=True) / (_B * _D)
    idx_ref[...] = jnp.concatenate(idx_cols, axis=1)
    loss_ref[...] += loss_acc


def kernel(x, codebook):
    t0, t1, t2, idx, loss = pl.pallas_call(
        _vq_body,
        grid=(_NBLK,),
        in_specs=[
            pl.BlockSpec((_BLK, _D), lambda i: (i, 0)),
            pl.BlockSpec((_K, _D), lambda i: (0, 0)),
            pl.BlockSpec((_D, _K), lambda i: (0, 0)),
        ],
        out_specs=[
            pl.BlockSpec((_BLK, _D), lambda i: (i, 0)),
            pl.BlockSpec((_BLK, _D), lambda i: (i, 0)),
            pl.BlockSpec((_BLK, _D), lambda i: (i, 0)),
            pl.BlockSpec((_BLK, _Q), lambda i: (i, 0)),
            pl.BlockSpec((1, 1), lambda i: (0, 0)),
        ],
        out_shape=[
            jax.ShapeDtypeStruct((_B, _D), jnp.float32),
            jax.ShapeDtypeStruct((_B, _D), jnp.float32),
            jax.ShapeDtypeStruct((_B, _D), jnp.float32),
            jax.ShapeDtypeStruct((_B, _Q), jnp.int32),
            jax.ShapeDtypeStruct((1, 1), jnp.float32),
        ],
        scratch_shapes=[
            pltpu.VMEM((_D, _K), jnp.bfloat16),
            pltpu.VMEM((_K, _D), jnp.float32),
        ],
        compiler_params=pltpu.CompilerParams(
            dimension_semantics=("arbitrary",)),
    )(x, codebook, codebook.T)
    tokens = jnp.stack([t0, t1, t2], axis=1)
    return tokens, idx, loss[0, 0]


# speedup vs baseline: 3.2937x; 3.2937x over previous
import jax
import jax.numpy as jnp
from jax.experimental import pallas as pl
from jax.experimental.pallas import tpu as pltpu

_B = 8192
_D = 64
_K = 8192
_Q = 3
_BETA = 0.25
_BLK = 256
_NBLK = _B // _BLK
_EPS = 1e-12


def _vq_body(x_ref, cb_ref, cbt_ref, t0_ref, t1_ref, t2_ref, idx_ref,
             loss_ref, et_ref, ec_ref):
    i = pl.program_id(0)

    @pl.when(i == 0)
    def _init():
        cb = cb_ref[...]
        nrm = jnp.sqrt(jnp.sum(cb * cb, axis=1, keepdims=True))
        en = cb / jnp.maximum(nrm, _EPS)
        e1 = en.astype(jnp.bfloat16)
        r1 = en - e1.astype(jnp.float32)
        e2 = r1.astype(jnp.bfloat16)
        r2 = r1 - e2.astype(jnp.float32)
        e3 = r2.astype(jnp.bfloat16)
        ec_ref[...] = jnp.concatenate([e1, e2, e3], axis=1)
        cbt = cbt_ref[...]
        nrm_t = jnp.sqrt(jnp.sum(cbt * cbt, axis=0, keepdims=True))
        et_ref[...] = (cbt / jnp.maximum(nrm_t, _EPS)).astype(jnp.bfloat16)
        loss_ref[...] = jnp.zeros((1, 1), jnp.float32)

    et = et_ref[...]
    r = x_ref[...]
    tok_refs = (t0_ref, t1_ref, t2_ref)
    idx_cols = []
    loss_acc = jnp.zeros((1, 1), jnp.float32)
    for q in range(_Q):
        rn = jnp.sqrt(jnp.sum(r * r, axis=1, keepdims=True))
        n = r / jnp.maximum(rn, _EPS)
        dist = jax.lax.dot_general(
            n.astype(jnp.bfloat16), et, (((1,), (0,)), ((), ())),
            preferred_element_type=jnp.float32)
        idx = jnp.argmax(dist, axis=1, keepdims=True).astype(jnp.int32)
        iota = jax.lax.broadcasted_iota(jnp.int32, (_BLK, _K), 1)
        onehot = (iota == idx).astype(jnp.bfloat16)
        q3 = jax.lax.dot_general(
            onehot, ec_ref[...], (((1,), (0,)), ((), ())),
            preferred_element_type=jnp.float32)
        quant = (q3[:, :_D] + q3[:, _D:2 * _D]) + q3[:, 2 * _D:]
        quantized = r + (quant - r)
        qn = jnp.sqrt(jnp.sum(quantized * quantized, axis=1, keepdims=True))
        c = quantized / jnp.maximum(qn, _EPS)
        cos = jnp.clip(jnp.sum(n * c, axis=1, keepdims=True), -1.0, 1.0)
        loss_acc += (_BETA / _B) * jnp.sum(1.0 - cos, keepdims=True)
        tok_refs[q][...] = r + (quantized - r)
        idx_cols.append(idx)
        alpha = jnp.sum(r * c, axis=1, keepdims=True)
        r = r - alpha * c
    loss_acc += jnp.sum(r * r, keepdims=True) / (_B * _D)
    idx_ref[...] = jnp.concatenate(idx_cols, axis=1)
    loss_ref[...] += loss_acc


def kernel(x, codebook):
    t0, t1, t2, idx, loss = pl.pallas_call(
        _vq_body,
        grid=(_NBLK,),
        in_specs=[
            pl.BlockSpec((_BLK, _D), lambda i: (i, 0)),
            pl.BlockSpec((_K, _D), lambda i: (0, 0)),
            pl.BlockSpec((_D, _K), lambda i: (0, 0)),
        ],
        out_specs=[
            pl.BlockSpec((_BLK, _D), lambda i: (i, 0)),
            pl.BlockSpec((_BLK, _D), lambda i: (i, 0)),
            pl.BlockSpec((_BLK, _D), lambda i: (i, 0)),
            pl.BlockSpec((_BLK, _Q), lambda i: (i, 0)),
            pl.BlockSpec((1, 1), lambda i: (0, 0)),
        ],
        out_shape=[
            jax.ShapeDtypeStruct((_B, _D), jnp.float32),
            jax.ShapeDtypeStruct((_B, _D), jnp.float32),
            jax.ShapeDtypeStruct((_B, _D), jnp.float32),
            jax.ShapeDtypeStruct((_B, _Q), jnp.int32),
            jax.ShapeDtypeStruct((1, 1), jnp.float32),
        ],
        scratch_shapes=[
            pltpu.VMEM((_D, _K), jnp.bfloat16),
            pltpu.VMEM((_K, 3 * _D), jnp.bfloat16),
        ],
        compiler_params=pltpu.CompilerParams(
            dimension_semantics=("arbitrary",)),
    )(x, codebook, codebook.T)
    tokens = jnp.stack([t0, t1, t2], axis=1)
    return tokens, idx, loss[0, 0]


# speedup vs baseline: 5.9784x; 1.8151x over previous
import jax
import jax.numpy as jnp
from jax.experimental import pallas as pl
from jax.experimental.pallas import tpu as pltpu

_B = 8192
_D = 64
_K = 8192
_Q = 3
_BETA = 0.25
_SUB = 256
_NSUB = 2
_BLK = _SUB * _NSUB
_NBLK = _B // _BLK
_EPS = 1e-12


def _vq_body(x_ref, cb_ref, cbt_ref, t0_ref, t1_ref, t2_ref, idx_ref,
             loss_ref, et_ref, ec_ref):
    i = pl.program_id(0)

    @pl.when(i == 0)
    def _init():
        cb = cb_ref[...]
        nrm = jnp.sqrt(jnp.sum(cb * cb, axis=1, keepdims=True))
        en = cb / jnp.maximum(nrm, _EPS)
        e1 = en.astype(jnp.bfloat16)
        r1 = en - e1.astype(jnp.float32)
        e2 = r1.astype(jnp.bfloat16)
        r2 = r1 - e2.astype(jnp.float32)
        e3 = r2.astype(jnp.bfloat16)
        ec_ref[...] = jnp.concatenate([e1, e2, e3], axis=1)
        cbt = cbt_ref[...]
        nrm_t = jnp.sqrt(jnp.sum(cbt * cbt, axis=0, keepdims=True))
        et_ref[...] = (cbt / jnp.maximum(nrm_t, _EPS)).astype(jnp.bfloat16)
        loss_ref[...] = jnp.zeros((1, 1), jnp.float32)

    et = et_ref[...]
    ec = ec_ref[...]
    iota = jax.lax.broadcasted_iota(jnp.int32, (_SUB, _K), 1)

    r = [x_ref[s * _SUB:(s + 1) * _SUB, :] for s in range(_NSUB)]
    n = [None] * _NSUB
    dist = [None] * _NSUB
    idx = [None] * _NSUB
    quant = [None] * _NSUB
    idx_cols = [[] for _ in range(_NSUB)]
    loss_acc = jnp.zeros((1, 1), jnp.float32)
    tok_refs = (t0_ref, t1_ref, t2_ref)

    for q in range(_Q):
        for s in range(_NSUB):
            rn = jnp.sqrt(jnp.sum(r[s] * r[s], axis=1, keepdims=True))
            n[s] = r[s] / jnp.maximum(rn, _EPS)
            dist[s] = jax.lax.dot_general(
                n[s].astype(jnp.bfloat16), et, (((1,), (0,)), ((), ())),
                preferred_element_type=jnp.float32)
        for s in range(_NSUB):
            idx[s] = jnp.argmax(dist[s], axis=1,
                                keepdims=True).astype(jnp.int32)
        for s in range(_NSUB):
            oh = (iota == idx[s]).astype(jnp.bfloat16)
            q3 = jax.lax.dot_general(
                oh, ec, (((1,), (0,)), ((), ())),
                preferred_element_type=jnp.float32)
            quant[s] = (q3[:, :_D] + q3[:, _D:2 * _D]) + q3[:, 2 * _D:]
        for s in range(_NSUB):
            quantized = r[s] + (quant[s] - r[s])
            qn = jnp.sqrt(jnp.sum(quantized * quantized,
                                  axis=1, keepdims=True))
            c = quantized / jnp.maximum(qn, _EPS)
            cos = jnp.clip(jnp.sum(n[s] * c, axis=1, keepdims=True),
                           -1.0, 1.0)
            loss_acc += (_BETA / _B) * jnp.sum(1.0 - cos, keepdims=True)
            tok_refs[q][s * _SUB:(s + 1) * _SUB, :] = (
                r[s] + (quantized - r[s]))
            idx_cols[s].append(idx[s])
            alpha = jnp.sum(r[s] * c, axis=1, keepdims=True)
            r[s] = r[s] - alpha * c
    for s in range(_NSUB):
        loss_acc += jnp.sum(r[s] * r[s], keepdims=True) / (_B * _D)
        idx_ref[s * _SUB:(s + 1) * _SUB, :] = jnp.concatenate(
            idx_cols[s], axis=1)
    loss_ref[...] += loss_acc


def kernel(x, codebook):
    t0, t1, t2, idx, loss = pl.pallas_call(
        _vq_body,
        grid=(_NBLK,),
        in_specs=[
            pl.BlockSpec((_BLK, _D), lambda i: (i, 0)),
            pl.BlockSpec((_K, _D), lambda i: (0, 0)),
            pl.BlockSpec((_D, _K), lambda i: (0, 0)),
        ],
        out_specs=[
            pl.BlockSpec((_BLK, _D), lambda i: (i, 0)),
            pl.BlockSpec((_BLK, _D), lambda i: (i, 0)),
            pl.BlockSpec((_BLK, _D), lambda i: (i, 0)),
            pl.BlockSpec((_BLK, _Q), lambda i: (i, 0)),
            pl.BlockSpec((1, 1), lambda i: (0, 0)),
        ],
        out_shape=[
            jax.ShapeDtypeStruct((_B, _D), jnp.float32),
            jax.ShapeDtypeStruct((_B, _D), jnp.float32),
            jax.ShapeDtypeStruct((_B, _D), jnp.float32),
            jax.ShapeDtypeStruct((_B, _Q), jnp.int32),
            jax.ShapeDtypeStruct((1, 1), jnp.float32),
        ],
        scratch_shapes=[
            pltpu.VMEM((_D, _K), jnp.bfloat16),
            pltpu.VMEM((_K, 3 * _D), jnp.bfloat16),
        ],
        compiler_params=pltpu.CompilerParams(
            dimension_semantics=("arbitrary",)),
    )(x, codebook, codebook.T)
    tokens = jnp.stack([t0, t1, t2], axis=1)
    return tokens, idx, loss[0, 0]


# speedup vs baseline: 6.6080x; 1.1053x over previous
import jax
import jax.numpy as jnp
from jax.experimental import pallas as pl
from jax.experimental.pallas import tpu as pltpu

_B = 8192
_D = 64
_K = 8192
_Q = 3
_BETA = 0.25
_SUB = 256
_NSUB = 4
_BLK = _SUB * _NSUB
_NBLK = _B // _BLK
_EPS = 1e-12


def _vq_body(x_ref, cb_ref, cbt_ref, t0_ref, t1_ref, t2_ref, idx_ref,
             loss_ref, et_ref, ec_ref):
    i = pl.program_id(0)

    @pl.when(i == 0)
    def _init():
        cb = cb_ref[...]
        nrm = jnp.sqrt(jnp.sum(cb * cb, axis=1, keepdims=True))
        en = cb / jnp.maximum(nrm, _EPS)
        e1 = en.astype(jnp.bfloat16)
        r1 = en - e1.astype(jnp.float32)
        e2 = r1.astype(jnp.bfloat16)
        r2 = r1 - e2.astype(jnp.float32)
        e3 = r2.astype(jnp.bfloat16)
        ec_ref[...] = jnp.concatenate([e1, e2, e3], axis=1)
        cbt = cbt_ref[...]
        nrm_t = jnp.sqrt(jnp.sum(cbt * cbt, axis=0, keepdims=True))
        et_ref[...] = (cbt / jnp.maximum(nrm_t, _EPS)).astype(jnp.bfloat16)
        loss_ref[...] = jnp.zeros((1, 1), jnp.float32)

    et = et_ref[...]
    ec = ec_ref[...]
    iota = jax.lax.broadcasted_iota(jnp.int32, (_SUB, _K), 1)

    r = [x_ref[s * _SUB:(s + 1) * _SUB, :] for s in range(_NSUB)]
    n = [None] * _NSUB
    dist = [None] * _NSUB
    idx = [None] * _NSUB
    quant = [None] * _NSUB
    idx_cols = [[] for _ in range(_NSUB)]
    loss_acc = jnp.zeros((1, 1), jnp.float32)
    tok_refs = (t0_ref, t1_ref, t2_ref)

    for q in range(_Q):
        for s in range(_NSUB):
            rn = jnp.sqrt(jnp.sum(r[s] * r[s], axis=1, keepdims=True))
            n[s] = r[s] / jnp.maximum(rn, _EPS)
            dist[s] = jax.lax.dot_general(
                n[s].astype(jnp.bfloat16), et, (((1,), (0,)), ((), ())),
                preferred_element_type=jnp.float32)
        for s in range(_NSUB):
            idx[s] = jnp.argmax(dist[s], axis=1,
                                keepdims=True).astype(jnp.int32)
        for s in range(_NSUB):
            oh = (iota == idx[s]).astype(jnp.bfloat16)
            q3 = jax.lax.dot_general(
                oh, ec, (((1,), (0,)), ((), ())),
                preferred_element_type=jnp.float32)
            quant[s] = (q3[:, :_D] + q3[:, _D:2 * _D]) + q3[:, 2 * _D:]
        for s in range(_NSUB):
            quantized = r[s] + (quant[s] - r[s])
            qn = jnp.sqrt(jnp.sum(quantized * quantized,
                                  axis=1, keepdims=True))
            c = quantized / jnp.maximum(qn, _EPS)
            cos = jnp.clip(jnp.sum(n[s] * c, axis=1, keepdims=True),
                           -1.0, 1.0)
            loss_acc += (_BETA / _B) * jnp.sum(1.0 - cos, keepdims=True)
            tok_refs[q][s * _SUB:(s + 1) * _SUB, :] = (
                r[s] + (quantized - r[s]))
            idx_cols[s].append(idx[s])
            alpha = jnp.sum(r[s] * c, axis=1, keepdims=True)
            r[s] = r[s] - alpha * c
    for s in range(_NSUB):
        loss_acc += jnp.sum(r[s] * r[s], keepdims=True) / (_B * _D)
        idx_ref[s * _SUB:(s + 1) * _SUB, :] = jnp.concatenate(
            idx_cols[s], axis=1)
    loss_ref[...] += loss_acc


def kernel(x, codebook):
    t0, t1, t2, idx, loss = pl.pallas_call(
        _vq_body,
        grid=(_NBLK,),
        in_specs=[
            pl.BlockSpec((_BLK, _D), lambda i: (i, 0)),
            pl.BlockSpec((_K, _D), lambda i: (0, 0)),
            pl.BlockSpec((_D, _K), lambda i: (0, 0)),
        ],
        out_specs=[
            pl.BlockSpec((_BLK, _D), lambda i: (i, 0)),
            pl.BlockSpec((_BLK, _D), lambda i: (i, 0)),
            pl.BlockSpec((_BLK, _D), lambda i: (i, 0)),
            pl.BlockSpec((_BLK, _Q), lambda i: (i, 0)),
            pl.BlockSpec((1, 1), lambda i: (0, 0)),
        ],
        out_shape=[
            jax.ShapeDtypeStruct((_B, _D), jnp.float32),
            jax.ShapeDtypeStruct((_B, _D), jnp.float32),
            jax.ShapeDtypeStruct((_B, _D), jnp.float32),
            jax.ShapeDtypeStruct((_B, _Q), jnp.int32),
            jax.ShapeDtypeStruct((1, 1), jnp.float32),
        ],
        scratch_shapes=[
            pltpu.VMEM((_D, _K), jnp.bfloat16),
            pltpu.VMEM((_K, 3 * _D), jnp.bfloat16),
        ],
        compiler_params=pltpu.CompilerParams(
            dimension_semantics=("arbitrary",)),
    )(x, codebook, codebook.T)
    tokens = jnp.stack([t0, t1, t2], axis=1)
    return tokens, idx, loss[0, 0]
